# baseline (speedup 1.0000x reference)
import functools

import jax
import jax.numpy as jnp
from jax import lax
from jax.experimental import pallas as pl
from jax.experimental.pallas import tpu as pltpu
from jax.experimental.pallas import tpu_sc as plsc

L = 16


def _pool_sc(inputs_flat, lengths, inv_lengths, table2, E):
    V2, E2 = table2.shape
    B = lengths.shape[0]
    S = inputs_flat.shape[0] // B
    EJ = E // L

    info = plsc.get_sparse_core_info()
    NC, NS = info.num_cores, info.num_subcores
    NW = NC * NS
    b_per_w = B // NW
    mesh = plsc.VectorSubcoreMesh(core_axis_name="c", subcore_axis_name="s")

    nbuf_rows = ((S + L - 1) // L) * L + L

    @functools.partial(
        pl.kernel, mesh=mesh,
        out_type=jax.ShapeDtypeStruct((B, E), jnp.float32),
        scratch_types=[
            pltpu.VMEM((b_per_w * S,), jnp.int32),
            pltpu.VMEM((b_per_w + L,), jnp.int32),
            pltpu.VMEM((b_per_w + L,), jnp.float32),
            pltpu.VMEM((nbuf_rows, E2), jnp.float32),
            pltpu.VMEM((nbuf_rows, E2), jnp.float32),
            pltpu.VMEM((b_per_w, E), jnp.float32),
            pltpu.SemaphoreType.DMA,
        ],
    )
    def k(inputs_hbm, lengths_hbm, inv_hbm, table_hbm, out_hbm,
          idx_v, len_v, inv_v, buf0, buf1, pool_v, sem):
        wid = lax.axis_index("s") * NC + lax.axis_index("c")
        base = wid * b_per_w
        pltpu.sync_copy(inputs_hbm.at[pl.ds(base * S, b_per_w * S)], idx_v)
        pltpu.sync_copy(lengths_hbm.at[pl.ds(base, b_per_w)],
                        len_v.at[pl.ds(0, b_per_w)])
        pltpu.sync_copy(inv_hbm.at[pl.ds(base, b_per_w)],
                        inv_v.at[pl.ds(0, b_per_w)])

        def nq_of(ln):
            return (ln + (L - 1)) // L

        def fire(b, ln, buf):
            def q_body(q, _):
                idx16 = idx_v[pl.ds(b * S + q * L, L)]
                pltpu.async_copy(table_hbm.at[jnp.right_shift(idx16, 1)],
                                 buf.at[pl.ds(q * L, L)], sem)
                return 0
            lax.fori_loop(0, nq_of(ln), q_body, 0)

        def drain(ln, buf):
            def q_body(q, _):
                pltpu.make_async_copy(
                    table_hbm.at[pl.ds(0, L)],
                    buf.at[pl.ds(q * L, L)], sem).wait()
                return 0
            lax.fori_loop(0, nq_of(ln), q_body, 0)

        def accumulate(b, ln, inv, buf):
            nfull = ln // L
            zeros = jnp.zeros((L,), jnp.float32)

            def full_body(g, accs):
                s0 = g * L
                par = jnp.bitwise_and(idx_v[pl.ds(b * S + s0, L)], 1) * E
                out = list(accs)
                for r in range(L):
                    off = par[r]
                    for j in range(EJ):
                        out[j] = out[j] + buf[s0 + r, pl.ds(off + j * L, L)]
                return tuple(out)

            accs = lax.fori_loop(
                0, nfull, full_body,
                tuple(jnp.zeros((L,), jnp.float32) for _ in range(EJ)))

            s0 = nfull * L
            par = jnp.bitwise_and(idx_v[pl.ds(b * S + s0, L)], 1) * E
            out = list(accs)
            for r in range(L):
                valid = (s0 + r) < ln
                off = par[r]
                for j in range(EJ):
                    row = buf[s0 + r, pl.ds(off + j * L, L)]
                    out[j] = out[j] + jnp.where(valid, row, zeros)

            for j in range(EJ):
                pool_v[b, pl.ds(j * L, L)] = out[j] * inv

        lens0 = len_v[pl.ds(0, L)]
        fire(0, lens0[0], buf0)

        def group_body(g, _):
            b0 = 8 * g
            lens = len_v[pl.ds(b0, L)]
            invs = inv_v[pl.ds(b0, L)]
            lens_next = len_v[pl.ds(b0 + 8, L)]
            for r in range(8):
                b = b0 + r
                ln = lens[r]
                nxt_ln = lens[r + 1] if r < 7 else lens_next[0]
                cur, nxt = (buf0, buf1) if r % 2 == 0 else (buf1, buf0)
                @pl.when(b + 1 < b_per_w)
                def _():
                    fire(b + 1, nxt_ln, nxt)
                drain(ln, cur)
                accumulate(b, ln, invs[r], cur)
            return 0
        lax.fori_loop(0, b_per_w // 8, group_body, 0)

        pltpu.sync_copy(pool_v, out_hbm.at[pl.ds(base, b_per_w)])

    return k(inputs_flat, lengths, inv_lengths, table2)


def _mlp_tc(pooled, W1, b1, W2, b2, W3, b3):
    B, E = pooled.shape
    H = W1.shape[1]
    C = W3.shape[1]
    BB = 512

    def body(x_ref, w1_ref, b1_ref, w2_ref, b2_ref, w3_ref, b3_ref, o_ref):
        x = x_ref[...]
        h = jnp.maximum(jnp.dot(x, w1_ref[...],
                                preferred_element_type=jnp.float32)
                        + b1_ref[...], 0.0)
        h = jnp.maximum(jnp.dot(h, w2_ref[...],
                                preferred_element_type=jnp.float32)
                        + b2_ref[...], 0.0)
        o_ref[...] = (jnp.dot(h, w3_ref[...],
                              preferred_element_type=jnp.float32)
                      + b3_ref[...])

    full = lambda shape: pl.BlockSpec(shape, lambda i: (0, 0))
    return pl.pallas_call(
        body,
        grid=(B // BB,),
        in_specs=[
            pl.BlockSpec((BB, E), lambda i: (i, 0)),
            full((E, H)), full((1, H)),
            full((H, H)), full((1, H)),
            full((H, C)), full((1, C)),
        ],
        out_specs=pl.BlockSpec((BB, C), lambda i: (i, 0)),
        out_shape=jax.ShapeDtypeStruct((B, C), jnp.float32),
    )(pooled, W1, b1.reshape(1, H), W2, b2.reshape(1, H),
      W3, b3.reshape(1, C))


def kernel(inputs, lengths, table, W1, b1, W2, b2, W3, b3):
    B, S = inputs.shape
    V, E = table.shape
    inputs_flat = inputs.astype(jnp.int32).reshape(B * S)
    lengths = lengths.astype(jnp.int32)
    inv_lengths = 1.0 / lengths.astype(jnp.float32)
    table2 = jnp.concatenate([table[0::2], table[1::2]], axis=1)
    pooled = _pool_sc(inputs_flat, lengths, inv_lengths, table2, E)
    return _mlp_tc(pooled, W1, b1, W2, b2, W3, b3)


# speedup vs baseline: 12.5687x; 12.5687x over previous
import functools

import jax
import jax.numpy as jnp
from jax import lax
from jax.experimental import pallas as pl
from jax.experimental.pallas import tpu as pltpu
from jax.experimental.pallas import tpu_sc as plsc

CH = 8
L = 16


def _pool_sc(inputs, lengths, inv_lengths, table):
    B, S = inputs.shape
    V, E = table.shape
    EJ = E // L

    info = plsc.get_sparse_core_info()
    NC, NS = info.num_cores, info.num_subcores
    NW = NC * NS
    b_per_w = B // NW
    mesh = plsc.VectorSubcoreMesh(core_axis_name="c", subcore_axis_name="s")

    nbuf_rows = ((S + L - 1) // L) * L + L

    @functools.partial(
        pl.kernel, mesh=mesh,
        compiler_params=pltpu.CompilerParams(use_tc_tiling_on_sc=False),
        out_type=jax.ShapeDtypeStruct((B, E), jnp.float32),
        scratch_types=[
            pltpu.VMEM((b_per_w, S), jnp.int32),
            pltpu.VMEM((b_per_w + L,), jnp.int32),
            pltpu.VMEM((b_per_w + L,), jnp.float32),
            pltpu.VMEM((nbuf_rows, E), jnp.float32),
            pltpu.VMEM((nbuf_rows, E), jnp.float32),
            pltpu.VMEM((b_per_w, E), jnp.float32),
            pltpu.SemaphoreType.DMA,
        ],
    )
    def k(inputs_hbm, lengths_hbm, inv_hbm, table_hbm, out_hbm,
          idx_v, len_v, inv_v, buf0, buf1, pool_v, sem):
        wid = lax.axis_index("s") * NC + lax.axis_index("c")
        base = wid * b_per_w
        pltpu.sync_copy(inputs_hbm.at[pl.ds(base, b_per_w)], idx_v)
        pltpu.sync_copy(lengths_hbm.at[pl.ds(base, b_per_w)],
                        len_v.at[pl.ds(0, b_per_w)])
        pltpu.sync_copy(inv_hbm.at[pl.ds(base, b_per_w)],
                        inv_v.at[pl.ds(0, b_per_w)])

        def ln_at(b):
            return len_v[pl.ds(b, L)][0]

        def nq_of(ln):
            return (ln + (CH - 1)) // CH

        def fire(b, buf):
            ln = ln_at(b)
            def q_body(q, _):
                pltpu.async_copy(
                    table_hbm.at[idx_v.at[b, pl.ds(q * CH, CH)]],
                    buf.at[pl.ds(q * CH, CH)], sem)
                return 0
            lax.fori_loop(0, nq_of(ln), q_body, 0)

        def drain(b, buf):
            ln = ln_at(b)
            def q_body(q, _):
                pltpu.make_async_copy(
                    table_hbm.at[pl.ds(0, CH)],
                    buf.at[pl.ds(q * CH, CH)], sem).wait()
                return 0
            lax.fori_loop(0, nq_of(ln), q_body, 0)

        def accumulate(b, buf):
            ln = ln_at(b)
            nfull = ln // L
            zeros = jnp.zeros((L,), jnp.float32)

            def full_body(g, accs):
                s0 = g * L
                out = list(accs)
                for r in range(L):
                    for j in range(EJ):
                        out[j] = out[j] + buf[s0 + r, pl.ds(j * L, L)]
                return tuple(out)

            accs = lax.fori_loop(
                0, nfull, full_body,
                tuple(jnp.zeros((L,), jnp.float32) for _ in range(EJ)))

            s0 = nfull * L
            out = list(accs)
            for r in range(L):
                valid = (s0 + r) < ln
                for j in range(EJ):
                    row = buf[s0 + r, pl.ds(j * L, L)]
                    out[j] = out[j] + jnp.where(valid, row, zeros)

            inv = inv_v[pl.ds(b, L)][0]
            for j in range(EJ):
                pool_v[b, pl.ds(j * L, L)] = out[j] * inv

        fire(0, buf0)
        def pair_body(i, _):
            b0 = 2 * i
            b1 = b0 + 1
            fire(b1, buf1)
            drain(b0, buf0)
            accumulate(b0, buf0)
            @pl.when(b0 + 2 < b_per_w)
            def _():
                fire(b0 + 2, buf0)
            drain(b1, buf1)
            accumulate(b1, buf1)
            return 0
        lax.fori_loop(0, b_per_w // 2, pair_body, 0)

        pltpu.sync_copy(pool_v, out_hbm.at[pl.ds(base, b_per_w)])

    return k(inputs, lengths, inv_lengths, table)


def _mlp_tc(pooled, W1, b1, W2, b2, W3, b3):
    B, E = pooled.shape
    H = W1.shape[1]
    C = W3.shape[1]
    BB = 512

    def body(x_ref, w1_ref, b1_ref, w2_ref, b2_ref, w3_ref, b3_ref, o_ref):
        x = x_ref[...]
        h = jnp.maximum(jnp.dot(x, w1_ref[...],
                                preferred_element_type=jnp.float32)
                        + b1_ref[...], 0.0)
        h = jnp.maximum(jnp.dot(h, w2_ref[...],
                                preferred_element_type=jnp.float32)
                        + b2_ref[...], 0.0)
        o_ref[...] = (jnp.dot(h, w3_ref[...],
                              preferred_element_type=jnp.float32)
                      + b3_ref[...])

    full = lambda shape: pl.BlockSpec(shape, lambda i: (0, 0))
    return pl.pallas_call(
        body,
        grid=(B // BB,),
        in_specs=[
            pl.BlockSpec((BB, E), lambda i: (i, 0)),
            full((E, H)), full((1, H)),
            full((H, H)), full((1, H)),
            full((H, C)), full((1, C)),
        ],
        out_specs=pl.BlockSpec((BB, C), lambda i: (i, 0)),
        out_shape=jax.ShapeDtypeStruct((B, C), jnp.float32),
    )(pooled, W1, b1.reshape(1, H), W2, b2.reshape(1, H),
      W3, b3.reshape(1, C))


def kernel(inputs, lengths, table, W1, b1, W2, b2, W3, b3):
    inputs = inputs.astype(jnp.int32)
    lengths = lengths.astype(jnp.int32)
    inv_lengths = 1.0 / lengths.astype(jnp.float32)
    pooled = _pool_sc(inputs, lengths, inv_lengths, table)
    return _mlp_tc(pooled, W1, b1, W2, b2, W3, b3)
